# baseline (device time: 38117 ns/iter reference)
import jax
import jax.numpy as jnp
from jax import lax
from jax.experimental import pallas as pl
from jax.experimental.pallas import tpu as pltpu

N_DEV = 8


def kernel(x, w_mat, scale_x, scale_w):
    m, k_shard = x.shape
    k, n = w_mat.shape
    m_out = m // N_DEV
    kb = k // N_DEV

    s = (scale_x[0] * scale_w[0]).reshape(1, 1)

    def body(x_ref, w_ref, s_ref, out_ref, comm_ref, send_sems, recv_sems,
             local_sem):
        my = lax.axis_index("i")

        barrier = pltpu.get_barrier_semaphore()
        for h in range(1, N_DEV):
            peer = lax.rem(my + h, N_DEV)
            pl.semaphore_signal(
                barrier, inc=1,
                device_id=(peer,), device_id_type=pl.DeviceIdType.MESH,
            )
        pl.semaphore_wait(barrier, N_DEV - 1)

        own = pltpu.make_async_copy(
            x_ref.at[pl.ds(my * m_out, m_out), :],
            comm_ref.at[my],
            local_sem,
        )
        own.start()

        rdmas = []
        for h in range(1, N_DEV):
            tgt = lax.rem(my + h, N_DEV)
            rdma = pltpu.make_async_remote_copy(
                src_ref=x_ref.at[pl.ds(tgt * m_out, m_out), :],
                dst_ref=comm_ref.at[my],
                send_sem=send_sems.at[h - 1],
                recv_sem=recv_sems.at[my],
                device_id=(tgt,),
                device_id_type=pl.DeviceIdType.MESH,
            )
            rdma.start()
            rdmas.append(rdma)

        own.wait()

        for h in range(1, N_DEV):
            src = lax.rem(my + h, N_DEV)
            recv = pltpu.make_async_remote_copy(
                src_ref=comm_ref.at[src],
                dst_ref=comm_ref.at[src],
                send_sem=send_sems.at[h - 1],
                recv_sem=recv_sems.at[src],
                device_id=(src,),
                device_id_type=pl.DeviceIdType.MESH,
            )
            recv.wait_recv()

        acc = jnp.dot(comm_ref[0], w_ref[pl.ds(0, kb), :],
                      preferred_element_type=jnp.int32)
        for j in range(1, N_DEV):
            acc = acc + jnp.dot(comm_ref[j], w_ref[pl.ds(j * kb, kb), :],
                                preferred_element_type=jnp.int32)

        for r in rdmas:
            r.wait_send()

        y = acc.astype(jnp.float32) * s_ref[0, 0]
        out_ref[:, :] = y / (1.0 + jnp.exp(-jnp.clip(y, -60.0, 60.0)))

    return pl.pallas_call(
        body,
        out_shape=jax.ShapeDtypeStruct((m_out, n), jnp.float32),
        in_specs=[
            pl.BlockSpec(memory_space=pltpu.VMEM),
            pl.BlockSpec(memory_space=pltpu.VMEM),
            pl.BlockSpec(memory_space=pltpu.SMEM),
        ],
        out_specs=pl.BlockSpec(memory_space=pltpu.VMEM),
        scratch_shapes=[
            pltpu.VMEM((N_DEV, m_out, k_shard), jnp.int8),
            pltpu.SemaphoreType.DMA((N_DEV - 1,)),
            pltpu.SemaphoreType.DMA((N_DEV,)),
            pltpu.SemaphoreType.DMA,
        ],
        compiler_params=pltpu.CompilerParams(collective_id=0),
    )(x, w_mat, s)


# device time: 31495 ns/iter; 1.2103x vs baseline; 1.2103x over previous
import jax
import jax.numpy as jnp
from jax import lax
from jax.experimental import pallas as pl
from jax.experimental.pallas import tpu as pltpu

N_DEV = 8


def kernel(x, w_mat, scale_x, scale_w):
    m, k_shard = x.shape
    k, n = w_mat.shape
    m_out = m // N_DEV
    kb = k // N_DEV

    s = (scale_x[0] * scale_w[0]).reshape(1, 1)

    def body(x_ref, w_ref, s_ref, out_ref, comm_ref, send_sems, recv_sems,
             local_sem):
        my = lax.axis_index("i")

        barrier = pltpu.get_barrier_semaphore()
        for h in range(1, N_DEV):
            peer = lax.rem(my + h, N_DEV)
            pl.semaphore_signal(
                barrier, inc=1,
                device_id=(peer,), device_id_type=pl.DeviceIdType.MESH,
            )
        pl.semaphore_wait(barrier, N_DEV - 1)

        own = pltpu.make_async_copy(
            x_ref.at[pl.ds(my * m_out, m_out), :],
            comm_ref.at[:, pl.ds(my * kb, kb)],
            local_sem,
        )
        own.start()

        rdmas = []
        for h in range(1, N_DEV):
            tgt = lax.rem(my + h, N_DEV)
            rdma = pltpu.make_async_remote_copy(
                src_ref=x_ref.at[pl.ds(tgt * m_out, m_out), :],
                dst_ref=comm_ref.at[:, pl.ds(my * kb, kb)],
                send_sem=send_sems.at[h - 1],
                recv_sem=recv_sems.at[my],
                device_id=(tgt,),
                device_id_type=pl.DeviceIdType.MESH,
            )
            rdma.start()
            rdmas.append(rdma)

        own.wait()

        acc = jnp.dot(comm_ref[:, pl.ds(my * kb, kb)],
                      w_ref[pl.ds(my * kb, kb), :],
                      preferred_element_type=jnp.int32)
        for h in range(1, N_DEV):
            src = lax.rem(my + (N_DEV - h), N_DEV)
            recv = pltpu.make_async_remote_copy(
                src_ref=comm_ref.at[:, pl.ds(src * kb, kb)],
                dst_ref=comm_ref.at[:, pl.ds(src * kb, kb)],
                send_sem=send_sems.at[h - 1],
                recv_sem=recv_sems.at[src],
                device_id=(src,),
                device_id_type=pl.DeviceIdType.MESH,
            )
            recv.wait_recv()
            acc = acc + jnp.dot(comm_ref[:, pl.ds(src * kb, kb)],
                                w_ref[pl.ds(src * kb, kb), :],
                                preferred_element_type=jnp.int32)

        for r in rdmas:
            r.wait_send()

        y = acc.astype(jnp.float32) * s_ref[0, 0]
        out_ref[:, :] = y / (1.0 + jnp.exp(-jnp.clip(y, -60.0, 60.0)))

    return pl.pallas_call(
        body,
        out_shape=jax.ShapeDtypeStruct((m_out, n), jnp.float32),
        in_specs=[
            pl.BlockSpec(memory_space=pltpu.VMEM),
            pl.BlockSpec(memory_space=pltpu.VMEM),
            pl.BlockSpec(memory_space=pltpu.SMEM),
        ],
        out_specs=pl.BlockSpec(memory_space=pltpu.VMEM),
        scratch_shapes=[
            pltpu.VMEM((m_out, k), jnp.int8),
            pltpu.SemaphoreType.DMA((N_DEV - 1,)),
            pltpu.SemaphoreType.DMA((N_DEV,)),
            pltpu.SemaphoreType.DMA,
        ],
        compiler_params=pltpu.CompilerParams(collective_id=0),
    )(x, w_mat, s)
